# Initial kernel scaffold; baseline (speedup 1.0000x reference)
import jax
import jax.numpy as jnp
from jax.experimental import pallas as pl


def kernel(predictions, priors):
    raise NotImplementedError("write your pallas kernel here")


# speedup vs baseline: 4.1736x; 4.1736x over previous
import jax
import jax.numpy as jnp
from jax.experimental import pallas as pl

_NMS_THRESHOLD = 0.5
_TOP_K = 400
_CONFIDENCE_THRESHOLD = 0.5
_KEEP_TOP_K = 200
_VAR0, _VAR1 = 0.1, 0.2
_NEG = -1e9
_NEGF = -3.0e38
_N_PAD = 5120


def _detect_body(pred_ref, pri_ref, ox1_ref, oy1_ref, ox2_ref, oy2_ref,
                 olab_ref, osc_ref):
    B = pred_ref.shape[1]
    N = pred_ref.shape[2]

    l0 = pred_ref[0]
    l1 = pred_ref[1]
    l2 = pred_ref[2]
    l3 = pred_ref[3]
    pcx = pri_ref[0:1, :]
    pcy = pri_ref[1:2, :]
    pw = pri_ref[2:3, :]
    ph = pri_ref[3:4, :]
    cx = pcx + l0 * _VAR0 * pw
    cy = pcy + l1 * _VAR0 * ph
    w = pw * jnp.exp(l2 * _VAR1)
    h = ph * jnp.exp(l3 * _VAR1)
    x1 = cx - w / 2.0
    y1 = cy - h / 2.0
    x2 = cx + w / 2.0
    y2 = cy + h / 2.0

    m = pred_ref[4]
    lab = jnp.zeros((B, N), jnp.float32)
    for c in range(1, 21):
        cc = pred_ref[4 + c]
        gt = cc > m
        m = jnp.where(gt, cc, m)
        lab = jnp.where(gt, jnp.float32(c), lab)
    masked = jnp.where(m > _CONFIDENCE_THRESHOLD, m, _NEG)

    iota_n = jax.lax.broadcasted_iota(jnp.int32, (B, N), 1)
    lane_k = jax.lax.broadcasted_iota(jnp.int32, (1, _TOP_K), 1)
    lane_o = jax.lax.broadcasted_iota(jnp.int32, (1, _KEEP_TOP_K), 1)

    def topk_body(t, carry):
        masked, sx1, sy1, sx2, sy2, slab, ssc = carry
        mx = jnp.max(masked, axis=1, keepdims=True)
        eq = masked == mx
        idx = jnp.min(jnp.where(eq, iota_n, N), axis=1, keepdims=True)
        onehot = iota_n == idx

        def sel(a):
            return jnp.sum(jnp.where(onehot, a, 0.0), axis=1, keepdims=True)

        tm = lane_k == t
        sx1 = jnp.where(tm, sel(x1), sx1)
        sy1 = jnp.where(tm, sel(y1), sy1)
        sx2 = jnp.where(tm, sel(x2), sx2)
        sy2 = jnp.where(tm, sel(y2), sy2)
        slab = jnp.where(tm, sel(lab), slab)
        ssc = jnp.where(tm, mx, ssc)
        masked = jnp.where(onehot, _NEGF, masked)
        return masked, sx1, sy1, sx2, sy2, slab, ssc

    z = jnp.zeros((B, _TOP_K), jnp.float32)
    carry = (masked, z, z, z, z, z, z)
    _, sx1, sy1, sx2, sy2, slab, ssc = jax.lax.fori_loop(
        0, _TOP_K, topk_body, carry)

    svalid = (ssc > _CONFIDENCE_THRESHOLD).astype(jnp.float32)
    area = (jnp.clip(sx2 - sx1, 0.0, None) *
            jnp.clip(sy2 - sy1, 0.0, None))

    def ext(mask, a):
        return jnp.sum(jnp.where(mask, a, 0.0), axis=1, keepdims=True)

    def nms_body(i, keep):
        oh = lane_k == i
        bx1 = ext(oh, sx1)
        by1 = ext(oh, sy1)
        bx2 = ext(oh, sx2)
        by2 = ext(oh, sy2)
        bar = ext(oh, area)
        ki = ext(oh, keep) * ext(oh, svalid)
        ltx = jnp.maximum(bx1, sx1)
        lty = jnp.maximum(by1, sy1)
        rbx = jnp.minimum(bx2, sx2)
        rby = jnp.minimum(by2, sy2)
        iw = jnp.clip(rbx - ltx, 0.0, None)
        ih = jnp.clip(rby - lty, 0.0, None)
        inter = iw * ih
        union = bar + area - inter
        iou = inter / jnp.maximum(union, 1e-9)
        sup = ((iou > _NMS_THRESHOLD) & (lane_k > i) & (ki > 0.5))
        keep = keep * (1.0 - sup.astype(jnp.float32))
        keep = jnp.where(lane_k == i, ki, keep)
        return keep

    keep = jax.lax.fori_loop(0, _TOP_K, nms_body,
                             jnp.ones((B, _TOP_K), jnp.float32))

    def scatter_body(i, carry):
        ox1, oy1, ox2, oy2, olab, osc = carry
        oh = lane_k == i
        yi = ext(oh, sy1)
        kpi = ext(oh, keep)
        less = (keep > 0.5) & ((sy1 < yi) | ((sy1 == yi) & (lane_k < i)))
        rank = jnp.sum(less.astype(jnp.int32), axis=1, keepdims=True)
        wm = (lane_o == rank) & (kpi > 0.5)
        ox1 = jnp.where(wm, ext(oh, sx1), ox1)
        oy1 = jnp.where(wm, yi, oy1)
        ox2 = jnp.where(wm, ext(oh, sx2), ox2)
        oy2 = jnp.where(wm, ext(oh, sy2), oy2)
        olab = jnp.where(wm, ext(oh, slab), olab)
        osc = jnp.where(wm, ext(oh, ssc), osc)
        return ox1, oy1, ox2, oy2, olab, osc

    zo = jnp.zeros((B, _KEEP_TOP_K), jnp.float32)
    ox1, oy1, ox2, oy2, olab, osc = jax.lax.fori_loop(
        0, _TOP_K, scatter_body, (zo, zo, zo, zo, zo, zo))

    ox1_ref[...] = ox1
    oy1_ref[...] = oy1
    ox2_ref[...] = ox2
    oy2_ref[...] = oy2
    olab_ref[...] = olab
    osc_ref[...] = osc


@jax.jit
def kernel(predictions, priors):
    B, N, C = predictions.shape
    pred_t = jnp.transpose(predictions, (2, 0, 1))
    pred_t = jnp.pad(pred_t, ((0, 0), (0, 0), (0, _N_PAD - N)),
                     constant_values=_NEG)
    pri_t = jnp.pad(priors.T, ((0, 0), (0, _N_PAD - N)))

    outs = pl.pallas_call(
        _detect_body,
        out_shape=[jax.ShapeDtypeStruct((B, _KEEP_TOP_K), jnp.float32)
                   for _ in range(6)],
    )(pred_t, pri_t)
    return jnp.stack(outs, axis=-1)
